# baseline (speedup 1.0000x reference)
import functools

import jax
import jax.numpy as jnp
from jax import lax
from jax.experimental import pallas as pl
from jax.experimental.pallas import tpu as pltpu
from jax.experimental.pallas import tpu_sc as plsc

_IDX_CHUNK = 128


def _gather_call(B, D, NC, NS):
    NW = NC * NS
    b_per_w = B // NW
    n_chunks = b_per_w // _IDX_CHUNK
    mesh = plsc.VectorSubcoreMesh(core_axis_name="c", subcore_axis_name="s")

    @functools.partial(
        pl.kernel,
        mesh=mesh,
        out_type=jax.ShapeDtypeStruct((NW, n_chunks, _IDX_CHUNK, D), jnp.float32),
        scratch_types=[
            pltpu.VMEM((n_chunks, _IDX_CHUNK), jnp.int32),
            pltpu.VMEM((n_chunks, _IDX_CHUNK, D), jnp.float32),
            pltpu.SemaphoreType.DMA,
        ],
        compiler_params=pltpu.CompilerParams(use_tc_tiling_on_sc=False),
    )
    def body(idx_hbm, table_hbm, out_hbm, idx_v, rows_v, sem):
        wid = lax.axis_index("s") * NC + lax.axis_index("c")
        pltpu.sync_copy(idx_hbm.at[wid], idx_v)
        copies = [
            pltpu.async_copy(table_hbm.at[idx_v.at[j]], rows_v.at[j], sem)
            for j in range(n_chunks)
        ]
        for c in copies:
            c.wait()
        pltpu.sync_copy(rows_v, out_hbm.at[wid])

    return body


def kernel(e, params):
    B = e.shape[0]
    V, D = params.shape
    info = plsc.get_sparse_core_info()
    NC, NS = info.num_cores, info.num_subcores
    NW = NC * NS
    idx = e.astype(jnp.int32).reshape(NW, (B // NW) // _IDX_CHUNK, _IDX_CHUNK)
    out = _gather_call(B, D, NC, NS)(idx, params)
    return out.reshape(B, D)


# speedup vs baseline: 1.0336x; 1.0336x over previous
import functools

import jax
import jax.numpy as jnp
from jax import lax
from jax.experimental import pallas as pl
from jax.experimental.pallas import tpu as pltpu
from jax.experimental.pallas import tpu_sc as plsc

_L = 16


def _gather_call(B, V, D, NC, NS):
    NW = NC * NS
    n = B // NW
    G = n // _L
    mesh = plsc.VectorSubcoreMesh(core_axis_name="c", subcore_axis_name="s")

    @functools.partial(
        pl.kernel,
        mesh=mesh,
        out_type=jax.ShapeDtypeStruct((B, D), jnp.float32),
        scratch_types=[
            pltpu.VMEM((G, _L), jnp.int32),
            pltpu.SemaphoreType.DMA,
        ],
        compiler_params=pltpu.CompilerParams(needs_layout_passes=False),
    )
    def body(idx_hbm, table_hbm, out_hbm, idx_v, sem):
        wid = lax.axis_index("s") * NC + lax.axis_index("c")
        base = wid * n
        pltpu.sync_copy(idx_hbm.at[wid], idx_v)
        iota = lax.iota(jnp.int32, _L)

        def group(g, _):
            vec = idx_v[g]
            for j in range(_L):
                i = jnp.sum(jnp.where(iota == j, vec, 0))
                pltpu.make_async_copy(
                    table_hbm.at[i], out_hbm.at[base + g * _L + j], sem
                ).start()
            return 0

        lax.fori_loop(0, G, group, 0)
        pltpu.make_async_copy(
            table_hbm.at[pl.ds(0, n)], out_hbm.at[pl.ds(base, n)], sem
        ).wait()

    return body


def kernel(e, params):
    B = e.shape[0]
    V, D = params.shape
    info = plsc.get_sparse_core_info()
    NC, NS = info.num_cores, info.num_subcores
    NW = NC * NS
    idx = e.astype(jnp.int32).reshape(NW, (B // NW) // _L, _L)
    return _gather_call(B, V, D, NC, NS)(idx, params)


# speedup vs baseline: 1.7279x; 1.6717x over previous
import functools

import jax
import jax.numpy as jnp
from jax import lax
from jax.experimental import pallas as pl
from jax.experimental.pallas import tpu as pltpu
from jax.experimental.pallas import tpu_sc as plsc

_L = 16


def _gather_call(B, V, D, NC, NS):
    NW = NC * NS
    n = B // NW
    G = n // _L
    mesh = plsc.VectorSubcoreMesh(core_axis_name="c", subcore_axis_name="s")

    @functools.partial(
        pl.kernel,
        mesh=mesh,
        out_type=jax.ShapeDtypeStruct((B, D), jnp.float32),
        scratch_types=[
            pltpu.VMEM((G, _L), jnp.int32),
            pltpu.VMEM((n, D), jnp.float32),
            pltpu.SemaphoreType.DMA,
        ],
        compiler_params=pltpu.CompilerParams(needs_layout_passes=False),
    )
    def body(idx_hbm, table_hbm, out_hbm, idx_v, rows_v, sem):
        wid = lax.axis_index("s") * NC + lax.axis_index("c")
        base = wid * n
        pltpu.sync_copy(idx_hbm.at[wid], idx_v)
        iota = lax.iota(jnp.int32, _L)

        def group(g, _):
            vec = idx_v[g]
            for j in range(_L):
                i = jnp.sum(jnp.where(iota == j, vec, 0))
                pltpu.make_async_copy(
                    table_hbm.at[i], rows_v.at[g * _L + j], sem
                ).start()
            return 0

        lax.fori_loop(0, G, group, 0)
        pltpu.make_async_copy(table_hbm.at[pl.ds(0, n)], rows_v, sem).wait()
        pltpu.sync_copy(rows_v, out_hbm.at[pl.ds(base, n)])

    return body


def kernel(e, params):
    B = e.shape[0]
    V, D = params.shape
    info = plsc.get_sparse_core_info()
    NC, NS = info.num_cores, info.num_subcores
    NW = NC * NS
    idx = e.astype(jnp.int32).reshape(NW, (B // NW) // _L, _L)
    return _gather_call(B, V, D, NC, NS)(idx, params)
